# Initial kernel scaffold; baseline (speedup 1.0000x reference)
import jax
import jax.numpy as jnp
from jax.experimental import pallas as pl


def kernel(x, vectors, Wfc, bfc):
    raise NotImplementedError("write your pallas kernel here")


# speedup vs baseline: 79.7519x; 79.7519x over previous
import jax
import jax.numpy as jnp
from jax.experimental import pallas as pl

HU = (-1000.0, -75.0, 0.0, 15.0, 25.0, 40.0, 50.0, 200.0, 1000.0)
NPTS = 9
P = 4
VPP = P * P * P
VD = 16
OUT = 768
BN = 512


def _u_kernel(vec_ref, wt_ref, u_ref):
    wt = wt_ref[...]
    for k in range(NPTS):
        vcol = vec_ref[:, k : k + 1]
        prod = vcol * wt
        uk = prod.reshape(VPP, VD, OUT).sum(axis=1)
        u_ref[k * VPP : (k + 1) * VPP, :] = uk


def _main_kernel(x_ref, u_ref, b_ref, o_ref):
    xb = x_ref[...]
    hats = []
    inv = [1.0 / (HU[k + 1] - HU[k]) for k in range(NPTS - 1)]
    hats.append(jnp.clip((HU[1] - xb) * inv[0], 0.0, 1.0))
    for k in range(1, NPTS - 1):
        up = (xb - HU[k - 1]) * inv[k - 1]
        dn = (HU[k + 1] - xb) * inv[k]
        hats.append(jnp.maximum(jnp.minimum(up, dn), 0.0))
    hats.append(jnp.clip((xb - HU[NPTS - 2]) * inv[NPTS - 2], 0.0, 1.0))
    a = jnp.concatenate(hats, axis=1)
    acc = jax.lax.dot_general(
        a, u_ref[...], (((1,), (0,)), ((), ())),
        preferred_element_type=jnp.float32,
    )
    o_ref[...] = acc + b_ref[...]


def kernel(x, vectors, Wfc, bfc):
    b, c, d, h, w = x.shape
    nd, nh, nw = d // P, h // P, w // P
    xp = x.reshape(b, nd, P, nh, P, nw, P)
    xp = xp.transpose(0, 1, 3, 5, 2, 4, 6).reshape(-1, VPP)
    n = xp.shape[0]

    vec_t = vectors.reshape(NPTS, VPP * VD).T
    wt = Wfc.T

    u = pl.pallas_call(
        _u_kernel,
        out_shape=jax.ShapeDtypeStruct((NPTS * VPP, OUT), jnp.float32),
    )(vec_t, wt)

    grid = (n // BN,)
    out = pl.pallas_call(
        _main_kernel,
        grid=grid,
        in_specs=[
            pl.BlockSpec((BN, VPP), lambda i: (i, 0)),
            pl.BlockSpec((NPTS * VPP, OUT), lambda i: (0, 0)),
            pl.BlockSpec((1, OUT), lambda i: (0, 0)),
        ],
        out_specs=pl.BlockSpec((BN, OUT), lambda i: (i, 0)),
        out_shape=jax.ShapeDtypeStruct((n, OUT), jnp.float32),
    )(xp, u, bfc.reshape(1, OUT))

    out = out.reshape(b, nd, nh, nw, OUT)
    return out.transpose(0, 4, 1, 2, 3)
